# baseline (speedup 1.0000x reference)
import functools

import jax
import jax.numpy as jnp
from jax.experimental import pallas as pl
from jax.experimental.pallas import tpu as pltpu

_LN_EPS = 1e-5


def _fused_kernel(perm_ref, f_ref, a_ref, w_in_ref, b_in_ref,
                  g_in_ref, be_in_ref, wp_ref, gw_ref, cbias_ref,
                  out_ref, *, G, H, N, d_in, d_model):
    f = f_ref[...]
    a = a_ref[...]
    w_top = w_in_ref[0:d_in, :]
    w_bot = w_in_ref[d_in:2 * d_in, :]
    y = (jnp.dot(f, w_top, preferred_element_type=jnp.float32)
         + jnp.dot(a, w_bot, preferred_element_type=jnp.float32)
         + b_in_ref[...])

    mu = jnp.mean(y, axis=-1, keepdims=True)
    var = jnp.mean(y * y, axis=-1, keepdims=True) - mu * mu
    y = (y - mu) * jax.lax.rsqrt(var + _LN_EPS) * g_in_ref[...] + be_in_ref[...]
    y = jnp.maximum(y, 0.0)

    inv_hd = 1.0 / (H * d_model)
    for g in range(G):
        obs = []
        s = None
        ss = None
        for h in range(H):
            i = g * H + h
            p = perm_ref[i]
            sf = jnp.dot(p, y[i * N:(i + 1) * N, :],
                         preferred_element_type=jnp.float32)
            ob = jax.lax.dot_general(p, sf, (((0,), (0,)), ((), ())),
                                     preferred_element_type=jnp.float32)
            obs.append(ob)
            rs = jnp.sum(ob, axis=-1, keepdims=True)
            rq = jnp.sum(ob * ob, axis=-1, keepdims=True)
            s = rs if s is None else s + rs
            ss = rq if ss is None else ss + rq

        mu2 = s * inv_hd
        var2 = ss * inv_hd - mu2 * mu2
        rstd = jax.lax.rsqrt(var2 + _LN_EPS)
        zp = None
        for h in range(H):
            t = jnp.dot(obs[h], wp_ref[h * d_model:(h + 1) * d_model, :],
                        preferred_element_type=jnp.float32)
            zp = t if zp is None else zp + t
        out_ref[g] = rstd * zp + (cbias_ref[...] - (mu2 * rstd) * gw_ref[...])


def kernel(perm, adj, features, appd, w_in, b_in, ln_in_g, ln_in_b,
           ln_out_g, ln_out_b, w_head, b_head):
    del adj
    B, H, N, _ = perm.shape
    d_in = features.shape[-1]
    d_model = w_in.shape[1]
    nclass = w_head.shape[1]

    G = min(8, B)
    nb = B // G

    p2 = perm.reshape(B * H, N, N)
    f2 = features.reshape(B * H * N, d_in)
    a2 = appd.reshape(B * H * N, d_in)

    wp = ln_out_g.reshape(-1, 1) * w_head
    gw = ln_out_g @ w_head
    cbias = ln_out_b @ w_head + b_head

    fused = functools.partial(_fused_kernel, G=G, H=H, N=N, d_in=d_in,
                              d_model=d_model)
    return pl.pallas_call(
        fused,
        out_shape=jax.ShapeDtypeStruct((B, N, nclass), jnp.float32),
        grid=(nb,),
        in_specs=[
            pl.BlockSpec((G * H, N, N), lambda s: (s, 0, 0)),
            pl.BlockSpec((G * H * N, d_in), lambda s: (s, 0)),
            pl.BlockSpec((G * H * N, d_in), lambda s: (s, 0)),
            pl.BlockSpec((2 * d_in, d_model), lambda s: (0, 0)),
            pl.BlockSpec((1, d_model), lambda s: (0, 0)),
            pl.BlockSpec((1, d_model), lambda s: (0, 0)),
            pl.BlockSpec((1, d_model), lambda s: (0, 0)),
            pl.BlockSpec((H * d_model, nclass), lambda s: (0, 0)),
            pl.BlockSpec((1, nclass), lambda s: (0, 0)),
            pl.BlockSpec((1, nclass), lambda s: (0, 0)),
        ],
        out_specs=pl.BlockSpec((G, N, nclass), lambda s: (s, 0, 0)),
        compiler_params=pltpu.CompilerParams(
            dimension_semantics=("parallel",)),
    )(p2, f2, a2, w_in, b_in, ln_in_g, ln_in_b, wp, gw, cbias)


# speedup vs baseline: 1.0821x; 1.0821x over previous
import functools

import jax
import jax.numpy as jnp
from jax.experimental import pallas as pl
from jax.experimental.pallas import tpu as pltpu

_LN_EPS = 1e-5


def _fused_kernel(perm_ref, f_ref, a_ref, w_in_ref, b_in_ref,
                  g_in_ref, be_in_ref, g_out_ref, be_out_ref,
                  w_head_ref, b_head_ref, out_ref, z_ref,
                  *, G, H, N, d_in, d_model):
    f = f_ref[...]
    a = a_ref[...]
    w_top = w_in_ref[0:d_in, :]
    w_bot = w_in_ref[d_in:2 * d_in, :]
    y = (jnp.dot(f, w_top, preferred_element_type=jnp.float32)
         + jnp.dot(a, w_bot, preferred_element_type=jnp.float32)
         + b_in_ref[...])

    mu = jnp.mean(y, axis=-1, keepdims=True)
    var = jnp.mean(y * y, axis=-1, keepdims=True) - mu * mu
    y = (y - mu) * jax.lax.rsqrt(var + _LN_EPS) * g_in_ref[...] + be_in_ref[...]
    y = jnp.maximum(y, 0.0)

    stats = []
    for g in range(G):
        s = None
        q = None
        for h in range(H):
            i = g * H + h
            p = perm_ref[i]
            sf = jnp.dot(p, y[i * N:(i + 1) * N, :],
                         preferred_element_type=jnp.float32)
            ob = jax.lax.dot_general(p, sf, (((0,), (0,)), ((), ())),
                                     preferred_element_type=jnp.float32)
            z_ref[g * N:(g + 1) * N, h * d_model:(h + 1) * d_model] = ob
            rs = jnp.sum(ob, axis=-1, keepdims=True)
            rq = jnp.sum(ob * ob, axis=-1, keepdims=True)
            s = rs if s is None else s + rs
            q = rq if q is None else q + rq
        stats.append((s, q))

    inv_hd = 1.0 / (H * d_model)
    mu2 = jnp.concatenate([s for s, _ in stats], axis=0) * inv_hd
    q2 = jnp.concatenate([q for _, q in stats], axis=0) * inv_hd
    rstd = jax.lax.rsqrt(q2 - mu2 * mu2 + _LN_EPS)

    z = z_ref[...]
    zn = (z - mu2) * rstd * g_out_ref[...] + be_out_ref[...]
    out_ref[...] = (jnp.dot(zn, w_head_ref[...],
                            preferred_element_type=jnp.float32)
                    + b_head_ref[...])


def kernel(perm, adj, features, appd, w_in, b_in, ln_in_g, ln_in_b,
           ln_out_g, ln_out_b, w_head, b_head):
    del adj
    B, H, N, _ = perm.shape
    d_in = features.shape[-1]
    d_model = w_in.shape[1]
    nclass = w_head.shape[1]

    G = min(8, B)
    nb = B // G

    p2 = perm.reshape(B * H, N, N)
    f2 = features.reshape(B * H * N, d_in)
    a2 = appd.reshape(B * H * N, d_in)

    fused = functools.partial(_fused_kernel, G=G, H=H, N=N, d_in=d_in,
                              d_model=d_model)
    out = pl.pallas_call(
        fused,
        out_shape=jax.ShapeDtypeStruct((B * N, nclass), jnp.float32),
        grid=(nb,),
        in_specs=[
            pl.BlockSpec((G * H, N, N), lambda s: (s, 0, 0)),
            pl.BlockSpec((G * H * N, d_in), lambda s: (s, 0)),
            pl.BlockSpec((G * H * N, d_in), lambda s: (s, 0)),
            pl.BlockSpec((2 * d_in, d_model), lambda s: (0, 0)),
            pl.BlockSpec((1, d_model), lambda s: (0, 0)),
            pl.BlockSpec((1, d_model), lambda s: (0, 0)),
            pl.BlockSpec((1, d_model), lambda s: (0, 0)),
            pl.BlockSpec((1, H * d_model), lambda s: (0, 0)),
            pl.BlockSpec((1, H * d_model), lambda s: (0, 0)),
            pl.BlockSpec((H * d_model, nclass), lambda s: (0, 0)),
            pl.BlockSpec((1, nclass), lambda s: (0, 0)),
        ],
        out_specs=pl.BlockSpec((G * N, nclass), lambda s: (s, 0)),
        scratch_shapes=[pltpu.VMEM((G * N, H * d_model), jnp.float32)],
        compiler_params=pltpu.CompilerParams(
            dimension_semantics=("parallel",)),
    )(p2, f2, a2, w_in, b_in, ln_in_g, ln_in_b,
      ln_out_g, ln_out_b, w_head, b_head)
    return out.reshape(B, N, nclass)
